# baseline (speedup 1.0000x reference)
import functools

import jax
import jax.numpy as jnp
from jax import lax
from jax.experimental import pallas as pl
from jax.experimental.pallas import tpu as pltpu
from jax.experimental.pallas import tpu_sc as plsc

_B = 256
_W = 4
_NCH = 4
_NCO = 3
_D = 300
_NWORD = 2010
_NCOMPO = 20000
_NC = 2
_NS = 16
_NW = _NC * _NS
_BPW = _B // _NW
_L = 16
_NFULL = _D // _L
_TAIL = _D - _NFULL * _L

_VB = 2048
_NA = -(-_NCOMPO // _VB)
_NB = 4
_NBLK = (_B * _W) // _NB


def _sc_body(widx_hbm, word_hbm, wctx_out, widx_v, wrows_v, wsem):
    wid = lax.axis_index("s") * _NC + lax.axis_index("c")
    nw = _BPW * _W

    pltpu.sync_copy(widx_hbm.at[pl.ds(wid * nw, nw)], widx_v)

    def _scalars(ref, n):
        vals = [None] * n
        starts = sorted({*range(0, n - _L + 1, _L), n - _L})
        for s in starts:
            v = ref[pl.ds(s, _L)]
            for l in range(_L):
                if vals[s + l] is None:
                    vals[s + l] = v[l]
        return vals

    wids = _scalars(widx_v, nw)

    wd = [pltpu.async_copy(word_hbm.at[pl.ds(wids[r], 1)],
                           wrows_v.at[pl.ds(r, 1)], wsem)
          for r in range(nw)]
    for d in wd:
        d.wait()
    pltpu.sync_copy(wrows_v, wctx_out.at[pl.ds(wid * nw, nw)])


@functools.lru_cache(maxsize=1)
def _get_sc_gather():
    return functools.partial(
        pl.kernel,
        out_type=jax.ShapeDtypeStruct((_B * _W, _D), jnp.float32),
        mesh=plsc.VectorSubcoreMesh(core_axis_name="c", subcore_axis_name="s"),
        scratch_types=[
            pltpu.VMEM((_BPW * _W,), jnp.int32),
            pltpu.VMEM((_BPW * _W, _D), jnp.float32),
            pltpu.SemaphoreType.DMA,
        ],
    )(_sc_body)


_NAC = -(-5000 // _VB)


def _onehot_accum(ids_ref, tbl_ref, nids, step, out_ref, first, ragged):
    v0 = step * _VB
    hb = _VB // 2
    cm = ids_ref[...]
    cma = jnp.where(cm == 1, -2, cm) - v0
    iota_v = lax.broadcasted_iota(jnp.int32, (hb, _B), 0)
    part = None
    for h in range(2):
        o = jnp.zeros((hb, _B), jnp.float32)
        for j in range(nids):
            ids = (cma[:, j] - h * hb)[None, :]
            o = o + (iota_v == ids).astype(jnp.float32)
        blk = tbl_ref[:, pl.ds(h * hb, hb)]
        if ragged:
            blk = jnp.where(jnp.isfinite(blk), blk, 0.0)
        p = lax.dot_general(blk.astype(jnp.bfloat16), o.astype(jnp.bfloat16),
                            (((1,), (0,)), ((), ())),
                            preferred_element_type=jnp.float32)
        part = p if part is None else part + p

    @pl.when(first)
    def _():
        out_ref[...] = part

    @pl.when(jnp.logical_not(first))
    def _():
        out_ref[...] = out_ref[...] + part


def _tca_body(cm_ref, ch_ref, compot_ref, chart_ref, out_ref):
    i = pl.program_id(0)

    @pl.when(i < _NA - 1)
    def _compo():
        _onehot_accum(cm_ref, compot_ref, _NCO, i, out_ref, i == 0, False)

    @pl.when(i == _NA - 1)
    def _compo_last():
        _onehot_accum(cm_ref, compot_ref, _NCO, i, out_ref, i == 0, True)

    @pl.when(jnp.logical_and(i >= _NA, i < _NA + _NAC - 1))
    def _char():
        _onehot_accum(ch_ref, chart_ref, _NCH, i - _NA, out_ref, i == 0,
                      False)

    @pl.when(i == _NA + _NAC - 1)
    def _char_last():
        _onehot_accum(ch_ref, chart_ref, _NCH, i - _NA, out_ref, i == 0,
                      True)


_tc_compo = pl.pallas_call(
    _tca_body,
    grid=(_NA + _NAC,),
    in_specs=[
        pl.BlockSpec((_B, _NCO), lambda i: (0, 0)),
        pl.BlockSpec((_B, _NCH), lambda i: (0, 0)),
        pl.BlockSpec((_D, _VB), lambda i: (0, jnp.minimum(i, _NA - 1))),
        pl.BlockSpec((_D, _VB),
                     lambda i: (0, jnp.clip(i - _NA, 0, _NAC - 1))),
    ],
    out_specs=pl.BlockSpec((_D, _B), lambda i: (0, 0)),
    out_shape=jax.ShapeDtypeStruct((_D, _B), jnp.float32),
)


def _tcb_body(cw_ref, tgtct_ref, wctx_ref, noise_ref, out_ref, tgtb_acc):
    i = pl.program_id(0)

    @pl.when(i == 0)
    def _start():
        tgtb = tgtct_ref[...].T.astype(jnp.bfloat16)
        tgtb_acc[...] = tgtb
        dfull = lax.dot_general(tgtb, wctx_ref[...].astype(jnp.bfloat16),
                                (((1,), (1,)), ((), ())),
                                preferred_element_type=jnp.float32)
        row = lax.broadcasted_iota(jnp.int32, (_B, _B * _W), 0)
        col = lax.broadcasted_iota(jnp.int32, (_B, _B * _W), 1)
        masked = jnp.where((col // _W) == row, dfull, 0.0)
        gsel = ((lax.broadcasted_iota(jnp.int32, (_B * _W, _W), 0) % _W) ==
                lax.broadcasted_iota(jnp.int32, (_B * _W, _W), 1)
                ).astype(jnp.bfloat16)
        dots = lax.dot_general(masked.astype(jnp.bfloat16), gsel,
                               (((1,), (0,)), ((), ())),
                               preferred_element_type=jnp.float32)
        mask = (cw_ref[...] != 1).astype(jnp.float32)
        dots = dots * mask
        sd = 1.0 / (1.0 + jnp.exp(-dots))
        out_ref[...] = jnp.sum(jnp.log(sd)).reshape(1, 1)

    nf = noise_ref[...].astype(jnp.bfloat16)
    s = -lax.dot_general(tgtb_acc[...], nf, (((1,), (1,)), ((), ())),
                         preferred_element_type=jnp.float32)
    sig = 1.0 / (1.0 + jnp.exp(-s))
    out_ref[...] = out_ref[...] + jnp.sum(jnp.log(sig + 1e-32))

    @pl.when(i == _NB - 1)
    def _fin():
        out_ref[...] = out_ref[...] * (-1.0 / _B)


_tc_loss = pl.pallas_call(
    _tcb_body,
    grid=(_NB,),
    in_specs=[
        pl.BlockSpec((_B, _W), lambda i: (0, 0)),
        pl.BlockSpec((_D, _B), lambda i: (0, 0)),
        pl.BlockSpec((_B * _W, _D), lambda i: (0, 0)),
        pl.BlockSpec((_NBLK, _D), lambda i: (i, 0)),
    ],
    out_specs=pl.BlockSpec((1, 1), lambda i: (0, 0)),
    out_shape=jax.ShapeDtypeStruct((1, 1), jnp.float32),
    scratch_shapes=[
        pltpu.VMEM((_B, _D), jnp.bfloat16),
    ],
)


def kernel(tgt_chars, tgt_compos, ctx_words, noise, word_emb, char_emb,
           compo_emb):
    widx = ctx_words.reshape(-1).astype(jnp.int32)
    wctx = _get_sc_gather()(widx, word_emb)
    noise2 = noise.reshape(_B * _W, _D).astype(jnp.int32)
    tgt_t = _tc_compo(tgt_compos.astype(jnp.int32),
                      tgt_chars.astype(jnp.int32), compo_emb.T, char_emb.T)
    loss2d = _tc_loss(ctx_words.astype(jnp.int32), tgt_t, wctx, noise2)
    return loss2d[0, 0]


# speedup vs baseline: 1.0475x; 1.0475x over previous
import functools

import jax
import jax.numpy as jnp
from jax import lax
from jax.experimental import pallas as pl
from jax.experimental.pallas import tpu as pltpu
from jax.experimental.pallas import tpu_sc as plsc

_B = 256
_W = 4
_NCH = 4
_NCO = 3
_D = 300
_NWORD = 2010
_NCOMPO = 20000
_NC = 2
_NS = 16
_NW = _NC * _NS
_BPW = _B // _NW
_L = 16
_NFULL = _D // _L
_TAIL = _D - _NFULL * _L

_VB = 4096
_NA = -(-_NCOMPO // _VB)
_NB = 4
_NBLK = (_B * _W) // _NB


def _sc_body(widx_hbm, word_hbm, wctx_out, widx_v, wrows_v, wsem):
    wid = lax.axis_index("s") * _NC + lax.axis_index("c")
    nw = _BPW * _W

    pltpu.sync_copy(widx_hbm.at[pl.ds(wid * nw, nw)], widx_v)

    def _scalars(ref, n):
        vals = [None] * n
        starts = sorted({*range(0, n - _L + 1, _L), n - _L})
        for s in starts:
            v = ref[pl.ds(s, _L)]
            for l in range(_L):
                if vals[s + l] is None:
                    vals[s + l] = v[l]
        return vals

    wids = _scalars(widx_v, nw)

    wd = [pltpu.async_copy(word_hbm.at[pl.ds(wids[r], 1)],
                           wrows_v.at[pl.ds(r, 1)], wsem)
          for r in range(nw)]
    for d in wd:
        d.wait()
    pltpu.sync_copy(wrows_v, wctx_out.at[pl.ds(wid * nw, nw)])


@functools.lru_cache(maxsize=1)
def _get_sc_gather():
    return functools.partial(
        pl.kernel,
        out_type=jax.ShapeDtypeStruct((_B * _W, _D), jnp.float32),
        mesh=plsc.VectorSubcoreMesh(core_axis_name="c", subcore_axis_name="s"),
        scratch_types=[
            pltpu.VMEM((_BPW * _W,), jnp.int32),
            pltpu.VMEM((_BPW * _W, _D), jnp.float32),
            pltpu.SemaphoreType.DMA,
        ],
    )(_sc_body)


_NAC = -(-5000 // _VB)


def _onehot_accum(ids_ref, tbl_ref, nids, step, out_ref, first):
    v0 = step * _VB
    iota_v = lax.broadcasted_iota(jnp.int32, (_VB, _B), 0)
    cm = ids_ref[...]
    cma = jnp.where(cm == 1, -2, cm) - v0
    o = jnp.zeros((_VB, _B), jnp.float32)
    for j in range(nids):
        ids = cma[:, j][None, :]
        o = o + (iota_v == ids).astype(jnp.float32)
    blk = tbl_ref[...]
    blk = jnp.where(jnp.isfinite(blk), blk, 0.0).astype(jnp.bfloat16)
    part = lax.dot_general(blk, o.astype(jnp.bfloat16),
                           (((1,), (0,)), ((), ())),
                           preferred_element_type=jnp.float32)

    @pl.when(first)
    def _():
        out_ref[...] = part

    @pl.when(jnp.logical_not(first))
    def _():
        out_ref[...] = out_ref[...] + part


def _tca_body(cm_ref, ch_ref, compot_ref, chart_ref, out_ref):
    i = pl.program_id(0)

    @pl.when(i < _NA)
    def _compo():
        _onehot_accum(cm_ref, compot_ref, _NCO, i, out_ref, i == 0)

    @pl.when(i >= _NA)
    def _char():
        _onehot_accum(ch_ref, chart_ref, _NCH, i - _NA, out_ref, i == 0)


_tc_compo = pl.pallas_call(
    _tca_body,
    grid=(_NA + _NAC,),
    in_specs=[
        pl.BlockSpec((_B, _NCO), lambda i: (0, 0)),
        pl.BlockSpec((_B, _NCH), lambda i: (0, 0)),
        pl.BlockSpec((_D, _VB), lambda i: (0, jnp.minimum(i, _NA - 1))),
        pl.BlockSpec((_D, _VB),
                     lambda i: (0, jnp.clip(i - _NA, 0, _NAC - 1))),
    ],
    out_specs=pl.BlockSpec((_D, _B), lambda i: (0, 0)),
    out_shape=jax.ShapeDtypeStruct((_D, _B), jnp.float32),
)


def _tcb_body(cw_ref, tgtct_ref, wctx_ref, noise_ref, out_ref, tgtb_acc):
    i = pl.program_id(0)

    @pl.when(i == 0)
    def _start():
        tgtb = tgtct_ref[...].T.astype(jnp.bfloat16)
        tgtb_acc[...] = tgtb
        dfull = lax.dot_general(tgtb, wctx_ref[...].astype(jnp.bfloat16),
                                (((1,), (1,)), ((), ())),
                                preferred_element_type=jnp.float32)
        row = lax.broadcasted_iota(jnp.int32, (_B, _B * _W), 0)
        col = lax.broadcasted_iota(jnp.int32, (_B, _B * _W), 1)
        masked = jnp.where((col // _W) == row, dfull, 0.0)
        gsel = ((lax.broadcasted_iota(jnp.int32, (_B * _W, _W), 0) % _W) ==
                lax.broadcasted_iota(jnp.int32, (_B * _W, _W), 1)
                ).astype(jnp.bfloat16)
        dots = lax.dot_general(masked.astype(jnp.bfloat16), gsel,
                               (((1,), (0,)), ((), ())),
                               preferred_element_type=jnp.float32)
        mask = (cw_ref[...] != 1).astype(jnp.float32)
        dots = dots * mask
        sd = 1.0 / (1.0 + jnp.exp(-dots))
        out_ref[...] = jnp.sum(jnp.log(sd)).reshape(1, 1)

    nf = noise_ref[...].astype(jnp.bfloat16)
    s = -lax.dot_general(tgtb_acc[...], nf, (((1,), (1,)), ((), ())),
                         preferred_element_type=jnp.float32)
    sig = 1.0 / (1.0 + jnp.exp(-s))
    out_ref[...] = out_ref[...] + jnp.sum(jnp.log(sig + 1e-32))

    @pl.when(i == _NB - 1)
    def _fin():
        out_ref[...] = out_ref[...] * (-1.0 / _B)


_tc_loss = pl.pallas_call(
    _tcb_body,
    grid=(_NB,),
    in_specs=[
        pl.BlockSpec((_B, _W), lambda i: (0, 0)),
        pl.BlockSpec((_D, _B), lambda i: (0, 0)),
        pl.BlockSpec((_B * _W, _D), lambda i: (0, 0)),
        pl.BlockSpec((_NBLK, _D), lambda i: (i, 0)),
    ],
    out_specs=pl.BlockSpec((1, 1), lambda i: (0, 0)),
    out_shape=jax.ShapeDtypeStruct((1, 1), jnp.float32),
    scratch_shapes=[
        pltpu.VMEM((_B, _D), jnp.bfloat16),
    ],
)


def kernel(tgt_chars, tgt_compos, ctx_words, noise, word_emb, char_emb,
           compo_emb):
    widx = ctx_words.reshape(-1).astype(jnp.int32)
    wctx = _get_sc_gather()(widx, word_emb)
    noise2 = noise.reshape(_B * _W, _D).astype(jnp.int32)
    tgt_t = _tc_compo(tgt_compos.astype(jnp.int32),
                      tgt_chars.astype(jnp.int32), compo_emb.T, char_emb.T)
    loss2d = _tc_loss(ctx_words.astype(jnp.int32), tgt_t, wctx, noise2)
    return loss2d[0, 0]
